# baseline (speedup 1.0000x reference)
import jax
import jax.numpy as jnp
from jax.experimental import pallas as pl
from jax.experimental.pallas import tpu as pltpu

_BR = 8192


def _dense_sum_kernel(x_ref, o_ref):
    xb = x_ref[...]
    bb, hh, ww = o_ref.shape
    ones = jnp.ones((xb.shape[1], 128), jnp.float32)
    z = jnp.dot(xb, ones, preferred_element_type=jnp.float32)
    zr = z.reshape(bb, hh // 8, 8, ww, 128)
    row = jax.lax.broadcasted_iota(jnp.int32, (ww, 128), 0)
    col = jax.lax.broadcasted_iota(jnp.int32, (ww, 128), 1)
    m = (row == col).astype(jnp.float32)
    t = jnp.sum(zr * m[None, None, None], axis=3)
    o_ref[...] = t.reshape(bb, hh, 128)[:, :, :ww]


def kernel(x):
    b, c, h, w = x.shape
    rows = b * h * w
    bb = _BR // (h * w)
    x2d = jnp.transpose(x, (0, 2, 3, 1)).reshape(rows, c)

    return pl.pallas_call(
        _dense_sum_kernel,
        out_shape=jax.ShapeDtypeStruct((b, h, w), x.dtype),
        grid=(rows // _BR,),
        in_specs=[pl.BlockSpec((_BR, c), lambda i: (i, 0))],
        out_specs=pl.BlockSpec((bb, h, w), lambda i: (i, 0, 0)),
        compiler_params=pltpu.CompilerParams(
            dimension_semantics=("parallel",),
            vmem_limit_bytes=64 * 1024 * 1024,
        ),
    )(x2d)


# speedup vs baseline: 1.1009x; 1.1009x over previous
import jax
import jax.numpy as jnp
from jax.experimental import pallas as pl
from jax.experimental.pallas import tpu as pltpu

_GH = 4


def _hwb_sum_kernel(x_ref, o_ref):
    b, gh, w, c = x_ref.shape
    q = gh * w
    z = jnp.dot(
        x_ref[...].reshape(b * q, c),
        jnp.ones((c, 128), jnp.float32),
        preferred_element_type=jnp.float32,
    )
    zv = z.reshape(b, q, 128)
    row = jax.lax.broadcasted_iota(jnp.int32, (q, 128), 0)
    col = jax.lax.broadcasted_iota(jnp.int32, (q, 128), 1)
    m = (row == col).astype(jnp.float32)
    d = jnp.sum(zv * m[None], axis=1)
    o_ref[...] = d.T.reshape(gh, w, b)


def kernel(x):
    b, c, h, w = x.shape
    x4 = jnp.transpose(x, (0, 2, 3, 1))

    out_hwb = pl.pallas_call(
        _hwb_sum_kernel,
        out_shape=jax.ShapeDtypeStruct((h, w, b), x.dtype),
        grid=(h // _GH,),
        in_specs=[pl.BlockSpec((b, _GH, w, c), lambda j: (0, j, 0, 0))],
        out_specs=pl.BlockSpec((_GH, w, b), lambda j: (j, 0, 0)),
        compiler_params=pltpu.CompilerParams(
            dimension_semantics=("parallel",),
            vmem_limit_bytes=64 * 1024 * 1024,
        ),
    )(x4)
    return jnp.transpose(out_hwb, (2, 0, 1))
